# baseline (speedup 1.0000x reference)
import functools

import jax
import jax.numpy as jnp
from jax import lax
from jax.experimental import pallas as pl
from jax.experimental.pallas import tpu as pltpu
from jax.experimental.pallas import tpu_sc as plsc

CHUNK = 128


def kernel(tokens, W_E):
    B, P = tokens.shape
    V, D = W_E.shape
    N = B * P

    info = plsc.get_sparse_core_info()
    NC, NS = info.num_cores, info.num_subcores
    NW = NC * NS
    assert N % (NW * CHUNK) == 0
    per_w = N // NW
    n_chunks = per_w // CHUNK

    flat_tok = tokens.reshape(N).astype(jnp.int32)

    mesh = plsc.VectorSubcoreMesh(core_axis_name="c", subcore_axis_name="s")

    @functools.partial(
        pl.kernel,
        mesh=mesh,
        compiler_params=pltpu.CompilerParams(use_tc_tiling_on_sc=False),
        out_type=jax.ShapeDtypeStruct((N, D), jnp.float32),
        scratch_types=[
            pltpu.VMEM((per_w,), jnp.int32),
            pltpu.VMEM((CHUNK, D), jnp.float32),
            pltpu.SemaphoreType.DMA,
        ],
    )
    def emb(tok_hbm, table_hbm, out_hbm, idx_v, row_v, gsem):
        wid = lax.axis_index("s") * NC + lax.axis_index("c")
        base = wid * per_w
        pltpu.sync_copy(tok_hbm.at[pl.ds(base, per_w)], idx_v)

        def body(j, c):
            idx = idx_v.at[pl.ds(j * CHUNK, CHUNK)]
            pltpu.async_copy(table_hbm.at[idx], row_v, gsem).wait()
            pltpu.sync_copy(row_v, out_hbm.at[pl.ds(base + j * CHUNK, CHUNK), :])
            return c

        lax.fori_loop(0, n_chunks, body, 0)

    out = emb(flat_tok, W_E)
    return out.reshape(B, P, D)


# speedup vs baseline: 1.0783x; 1.0783x over previous
import functools

import jax
import jax.numpy as jnp
from jax import lax
from jax.experimental import pallas as pl
from jax.experimental.pallas import tpu as pltpu
from jax.experimental.pallas import tpu_sc as plsc


def _chunk_size(P):
    for c in range(min(P, 128), 7, -1):
        if P % c == 0 and c % 8 == 0:
            return c
    raise ValueError(f"no valid chunk size for P={P}")


def kernel(tokens, W_E):
    B, P = tokens.shape
    V, D = W_E.shape

    info = plsc.get_sparse_core_info()
    NC, NS = info.num_cores, info.num_subcores
    NW = NC * NS
    assert B % (2 * NW) == 0
    RPW = B // NW
    CH = _chunk_size(P)
    NCH = P // CH

    tokens = tokens.astype(jnp.int32)
    mesh = plsc.VectorSubcoreMesh(core_axis_name="c", subcore_axis_name="s")

    @functools.partial(
        pl.kernel,
        mesh=mesh,
        compiler_params=pltpu.CompilerParams(use_tc_tiling_on_sc=False),
        out_type=jax.ShapeDtypeStruct((B, P, D), jnp.float32),
        scratch_types=[
            pltpu.VMEM((RPW, P), jnp.int32),
            pltpu.VMEM((2, P, D), jnp.float32),
            pltpu.SemaphoreType.DMA,
            pltpu.SemaphoreType.DMA,
            pltpu.SemaphoreType.DMA,
            pltpu.SemaphoreType.DMA,
        ],
    )
    def emb(tok_hbm, table_hbm, out_hbm, idx_v, rows_v, g0, g1, o0, o1):
        wid = lax.axis_index("s") * NC + lax.axis_index("c")
        base = wid * RPW
        pltpu.sync_copy(tok_hbm.at[pl.ds(base, RPW), :], idx_v)

        def fire_gathers(r, s, gsem):
            for c in range(NCH):
                idx = idx_v.at[r, pl.ds(c * CH, CH)]
                pltpu.async_copy(table_hbm.at[idx], rows_v.at[s, pl.ds(c * CH, CH), :], gsem)

        def drain_gathers(gsem):
            pltpu.make_async_copy(table_hbm.at[pl.ds(0, P), :], rows_v.at[0], gsem).wait()

        def start_store(r, s, osem):
            pltpu.async_copy(rows_v.at[s], out_hbm.at[base + r], osem)

        def drain_store(osem):
            pltpu.make_async_copy(rows_v.at[0], out_hbm.at[0], osem).wait()

        fire_gathers(0, 0, g0)

        def body(rp, carry):
            r0 = 2 * rp
            drain_gathers(g0)
            start_store(r0, 0, o0)
            @pl.when(rp >= 1)
            def _():
                drain_store(o1)
            fire_gathers(r0 + 1, 1, g1)
            drain_gathers(g1)
            start_store(r0 + 1, 1, o1)
            @pl.when(rp < RPW // 2 - 1)
            def _():
                drain_store(o0)
                fire_gathers(r0 + 2, 0, g0)
            return carry

        lax.fori_loop(0, RPW // 2, body, 0)
        drain_store(o0)
        drain_store(o1)

    return emb(tokens, W_E)
